# baseline (speedup 1.0000x reference)
import functools

import jax
import jax.numpy as jnp
from jax import lax
from jax.experimental import pallas as pl
from jax.experimental.pallas import tpu as pltpu
from jax.experimental.pallas import tpu_sc as plsc

N = 10000
NPAD = 10240
E = 320000
EPAD = 327680
D = 128
NCH = 2
NET = 2
NCLASS = 16
NT = 2000
NTPAD = 2048
BETA = 0.5

NC = 2
NS = 16
NW = NC * NS
CH = 80
EROWS = EPAD // CH
ROWS_PER_W = EROWS // NW
GRP = 32
NGRP = ROWS_PER_W // GRP
ROWS_PER_T = EROWS // NS
NGRP_DEG = ROWS_PER_T // GRP
NSLICE = NPAD // NS


def _mesh():
    return plsc.VectorSubcoreMesh(core_axis_name="c", subcore_axis_name="s")


def _proj_body(x_ref, ws_ref, f0_ref, f1_ref, h_ref, scl_ref):
    ci = pl.program_id(0)
    bi = pl.program_id(1)
    h_ref[0] = jnp.dot(x_ref[...], ws_ref[0], preferred_element_type=jnp.float32)

    @pl.when(jnp.logical_and(ci == 0, bi == 0))
    def _():
        for l, fr in enumerate((f0_ref, f1_ref)):
            fw = jax.nn.softmax(fr[...], axis=1)
            scl_ref[l] = jnp.broadcast_to(fw[:, :, None], (NCH, NET, 16))


def _proj(xpad, Ws, filt0, filt1):
    BN = 1280
    grid = (NCH, NPAD // BN)
    return pl.pallas_call(
        _proj_body,
        grid=grid,
        in_specs=[
            pl.BlockSpec((BN, D), lambda c, i: (i, 0)),
            pl.BlockSpec((1, D, D), lambda c, i: (c, 0, 0)),
            pl.BlockSpec((NCH, NET), lambda c, i: (0, 0)),
            pl.BlockSpec((NCH, NET), lambda c, i: (0, 0)),
        ],
        out_specs=[
            pl.BlockSpec((1, BN, D), lambda c, i: (c, i, 0)),
            pl.BlockSpec((2, NCH, NET, 16), lambda c, i: (0, 0, 0, 0)),
        ],
        out_shape=[
            jax.ShapeDtypeStruct((NCH, NPAD, D), jnp.float32),
            jax.ShapeDtypeStruct((2, NCH, NET, 16), jnp.float32),
        ],
    )(xpad, Ws, filt0, filt1)


def _norm_body(r0, v0, r1, v1, zflat, evn0, evn1,
               rowb, evb, deg0, deg1, sumb, invb, inv0, inv1, evnb,
               spdeg, spinv):
    c = lax.axis_index("c")
    s = lax.axis_index("s")
    wid = s * NC + c

    for j, (rref, vref, dref) in enumerate(((r0, v0, deg0), (r1, v1, deg1))):
        pltpu.sync_copy(zflat.at[pl.ds(0, NPAD)], dref)

        def grp_body(g, _, rref=rref, vref=vref, dref=dref):
            base = s * ROWS_PER_T + g * GRP
            pltpu.sync_copy(rref.at[pl.ds(base, GRP)], rowb)
            pltpu.sync_copy(vref.at[pl.ds(base, GRP)], evb)

            def row_body(r, _):
                for k in range(CH // 16):
                    rv = rowb[r, pl.ds(k * 16, 16)]
                    ev = evb[r, pl.ds(k * 16, 16)]
                    plsc.addupdate_scatter(dref, [rv], ev)
                return 0

            lax.fori_loop(0, GRP, row_body, 0)
            return 0

        lax.fori_loop(0, NGRP_DEG, grp_body, 0)
        pltpu.sync_copy(dref, spdeg.at[j, s, 0])

    plsc.subcore_barrier()

    for j in range(NET):
        pltpu.sync_copy(spdeg.at[j, :, 0, pl.ds(s * NSLICE, NSLICE)], sumb)

        def inv_body(g, _):
            acc = sumb[0, pl.ds(g * 16, 16)]
            for r in range(1, NS):
                acc = acc + sumb[r, pl.ds(g * 16, 16)]
            inv = jnp.where(acc > 0.0, 1.0 / acc, 0.0)
            invb[pl.ds(g * 16, 16)] = inv
            return 0

        lax.fori_loop(0, NSLICE // 16, inv_body, 0)
        pltpu.sync_copy(invb, spinv.at[j, pl.ds(s * NSLICE, NSLICE)])

    plsc.subcore_barrier()
    pltpu.sync_copy(spinv.at[0], inv0)
    pltpu.sync_copy(spinv.at[1], inv1)

    for rref, vref, iref, oref in ((r0, v0, inv0, evn0), (r1, v1, inv1, evn1)):
        def grp2_body(g, _, rref=rref, vref=vref, iref=iref, oref=oref):
            base = wid * ROWS_PER_W + g * GRP
            pltpu.sync_copy(rref.at[pl.ds(base, GRP)], rowb)
            pltpu.sync_copy(vref.at[pl.ds(base, GRP)], evb)

            def row_body(r, _):
                for k in range(CH // 16):
                    rv = rowb[r, pl.ds(k * 16, 16)]
                    ev = evb[r, pl.ds(k * 16, 16)]
                    iv = plsc.load_gather(iref, [rv])
                    evnb[r, pl.ds(k * 16, 16)] = ev * iv
                return 0

            lax.fori_loop(0, GRP, row_body, 0)
            pltpu.sync_copy(evnb, oref.at[pl.ds(base, GRP)])
            return 0

        lax.fori_loop(0, NGRP, grp2_body, 0)


def _norm(r0, v0, r1, v1, zflat):
    kfn = pl.kernel(
        _norm_body,
        out_type=[
            jax.ShapeDtypeStruct((EROWS, CH), jnp.float32),
            jax.ShapeDtypeStruct((EROWS, CH), jnp.float32),
        ],
        mesh=_mesh(),
        compiler_params=pltpu.CompilerParams(needs_layout_passes=False),
        scratch_types=[
            pltpu.VMEM((GRP, CH), jnp.int32),
            pltpu.VMEM((GRP, CH), jnp.float32),
            pltpu.VMEM((NPAD,), jnp.float32),
            pltpu.VMEM((NPAD,), jnp.float32),
            pltpu.VMEM((NS, NSLICE), jnp.float32),
            pltpu.VMEM((NSLICE,), jnp.float32),
            pltpu.VMEM((NPAD,), jnp.float32),
            pltpu.VMEM((NPAD,), jnp.float32),
            pltpu.VMEM((GRP, CH), jnp.float32),
            pltpu.VMEM_SHARED((NET, NS, 1, NPAD), jnp.float32),
            pltpu.VMEM_SHARED((NET, NPAD), jnp.float32),
        ],
    )
    return kfn(r0, v0, r1, v1, zflat)


NBUF = 2


def _layer_body(hsrc, r0, c0, w0, r1, c1, w1, scl, z2d, p_out,
                idxr, idxc, wb, wsf, rows, svec, acc, gsem, ssem):
    c = lax.axis_index("c")
    s = lax.axis_index("s")
    wid = s * NC + c

    def g_desc(ch, k, slot):
        return pltpu.make_async_copy(
            hsrc.at[ch].at[idxc.at[k]], rows.at[slot], gsem.at[slot])

    def s_desc(k, slot):
        return pltpu.make_async_copy(
            rows.at[slot], acc.at[idxr.at[k]], ssem.at[slot])

    for ch in range(NCH):
        pltpu.sync_copy(z2d.at[pl.ds(s * NSLICE, NSLICE)],
                        acc.at[pl.ds(s * NSLICE, NSLICE)])
        plsc.subcore_barrier()

        for j, (rref, cref, wref) in enumerate(((r0, c0, w0), (r1, c1, w1))):
            pltpu.sync_copy(scl.at[ch, j], svec)

            def grp_body(g, _, rref=rref, cref=cref, wref=wref, ch=ch):
                base = wid * ROWS_PER_W + g * GRP
                pltpu.sync_copy(rref.at[pl.ds(base, GRP)], idxr)
                pltpu.sync_copy(cref.at[pl.ds(base, GRP)], idxc)
                pltpu.sync_copy(wref.at[pl.ds(base, GRP)], wb)

                def swr_body(r, _):
                    for k in range(CH // 16):
                        wsf[pl.ds(r * CH + k * 16, 16)] = (
                            wb[r, pl.ds(k * 16, 16)] * svec[...]
                        )
                    return 0

                lax.fori_loop(0, GRP, swr_body, 0)

                g_desc(ch, 0, 0).start()

                def pair_body(kk, _, ch=ch):
                    for i in range(NBUF):
                        k = NBUF * kk + i
                        oi = 1 - i

                        @pl.when(k + 1 < GRP)
                        def _(k=k, oi=oi):
                            @pl.when(k >= 1)
                            def _():
                                s_desc(k, oi).wait()
                            g_desc(ch, k + 1, oi).start()

                        g_desc(ch, k, i).wait()

                        def edge_body(e, _, k=k, i=i):
                            widx = jnp.full((16,), k * CH + e, jnp.int32)
                            w16 = plsc.load_gather(wsf, [widx])
                            for sub in range(D // 16):
                                rows[i, e, pl.ds(sub * 16, 16)] = (
                                    rows[i, e, pl.ds(sub * 16, 16)] * w16
                                )
                            return 0

                        lax.fori_loop(0, CH, edge_body, 0)
                        pltpu.async_copy(rows.at[i], acc.at[idxr.at[k]],
                                         ssem.at[i], add=True)
                    return 0

                lax.fori_loop(0, GRP // NBUF, pair_body, 0)
                s_desc(GRP - 2, 0).wait()
                s_desc(GRP - 1, 1).wait()
                return 0

            lax.fori_loop(0, NGRP, grp_body, 0)

        plsc.subcore_barrier()
        pltpu.sync_copy(acc.at[pl.ds(s * NSLICE, NSLICE)],
                        p_out.at[c, ch, pl.ds(s * NSLICE, NSLICE)])
        plsc.subcore_barrier()


def _layer(hsrc, r0, c0, w0, r1, c1, w1, scl, z2d):
    kfn = pl.kernel(
        _layer_body,
        out_type=jax.ShapeDtypeStruct((NC, NCH, NPAD, D), jnp.float32),
        mesh=_mesh(),
        compiler_params=pltpu.CompilerParams(needs_layout_passes=False),
        scratch_types=[
            pltpu.VMEM((GRP, CH), jnp.int32),
            pltpu.VMEM((GRP, CH), jnp.int32),
            pltpu.VMEM((GRP, CH), jnp.float32),
            pltpu.VMEM((GRP * CH,), jnp.float32),
            pltpu.VMEM((NBUF, CH, D), jnp.float32),
            pltpu.VMEM((16,), jnp.float32),
            pltpu.VMEM_SHARED((NPAD, D), jnp.float32),
            pltpu.SemaphoreType.DMA((NBUF,)),
            pltpu.SemaphoreType.DMA((NBUF,)),
        ],
    )
    return kfn(hsrc, r0, c0, w0, r1, c1, w1, scl, z2d)


CAPC = ROWS_PER_W
CAPE = CAPC * CH
GRP2 = 8


def _filter_body(r0, c0, w0, r1, c1, w1, tgt1d,
                 fr0, fc0, fw0, fr1, fc1, fw1, cnts,
                 flag, tball, rowb, colb, wvb, sbr, sbc, sbw, cntb):
    c = lax.axis_index("c")
    s = lax.axis_index("s")
    wid = s * NC + c
    ones = jnp.ones((16,), jnp.float32)

    def z_body(i, _):
        flag[pl.ds(i * 16, 16)] = jnp.zeros((16,), jnp.float32)
        return 0

    lax.fori_loop(0, NPAD // 16, z_body, 0)
    pltpu.sync_copy(tgt1d, tball)

    def t_body(i, _):
        tv = tball[pl.ds(i * 16, 16)]
        plsc.store_scatter(flag, [tv], ones)
        return 0

    lax.fori_loop(0, NTPAD // 16, t_body, 0)

    dumr = jnp.full((16,), N, jnp.int32)
    zi = jnp.zeros((16,), jnp.int32)
    zf = jnp.zeros((16,), jnp.float32)

    for j, (rref, cref, wref, orf, ocf, owf) in enumerate((
            (r0, c0, w0, fr0, fc0, fw0), (r1, c1, w1, fr1, fc1, fw1))):
        obase = wid * CAPE

        def grp_body(g, carry, rref=rref, cref=cref, wref=wref,
                     orf=orf, ocf=ocf, owf=owf, obase=obase):
            woff, nch = carry
            base = wid * ROWS_PER_W + g * GRP
            pltpu.sync_copy(rref.at[pl.ds(base, GRP)], rowb)
            pltpu.sync_copy(cref.at[pl.ds(base, GRP)], colb)
            pltpu.sync_copy(wref.at[pl.ds(base, GRP)], wvb)

            def row_body(r, woff):
                for k in range(CH // 16):
                    rv = rowb[r, pl.ds(k * 16, 16)]
                    cv = colb[r, pl.ds(k * 16, 16)]
                    wv = wvb[r, pl.ds(k * 16, 16)]
                    fv = plsc.load_gather(flag, [rv])
                    m = fv > 0.5
                    plsc.store_compressed(sbr.at[pl.ds(woff, 16)], rv, mask=m)
                    plsc.store_compressed(sbc.at[pl.ds(woff, 16)], cv, mask=m)
                    plsc.store_compressed(sbw.at[pl.ds(woff, 16)], wv, mask=m)
                    pc = plsc.all_reduce_population_count(m)
                    woff = woff + jnp.max(pc, axis=0)
                return woff

            woff = lax.fori_loop(0, GRP, row_body, woff)

            ndump = woff // CH

            def dump_body(d, _):
                off = obase + (nch + d) * CH
                pltpu.sync_copy(sbr.at[pl.ds(d * CH, CH)], orf.at[pl.ds(off, CH)])
                pltpu.sync_copy(sbc.at[pl.ds(d * CH, CH)], ocf.at[pl.ds(off, CH)])
                pltpu.sync_copy(sbw.at[pl.ds(d * CH, CH)], owf.at[pl.ds(off, CH)])
                return 0

            lax.fori_loop(0, ndump, dump_body, 0)
            rem = woff - ndump * CH

            for i in range(CH // 16 + 1):
                sr = sbr[pl.ds(ndump * CH + i * 16, 16)]
                sc_ = sbc[pl.ds(ndump * CH + i * 16, 16)]
                sw = sbw[pl.ds(ndump * CH + i * 16, 16)]
                sbr[pl.ds(i * 16, 16)] = sr
                sbc[pl.ds(i * 16, 16)] = sc_
                sbw[pl.ds(i * 16, 16)] = sw
            return (rem, nch + ndump)

        woff, nch = lax.fori_loop(0, NGRP, grp_body, (0, 0))

        for i in range(CH // 16 + 1):
            sbr[pl.ds(woff + i * 16, 16)] = dumr
            sbc[pl.ds(woff + i * 16, 16)] = zi
            sbw[pl.ds(woff + i * 16, 16)] = zf

        @pl.when(woff > 0)
        def _(nch=nch, obase=obase, orf=orf, ocf=ocf, owf=owf):
            off = obase + nch * CH
            pltpu.sync_copy(sbr.at[pl.ds(0, CH)], orf.at[pl.ds(off, CH)])
            pltpu.sync_copy(sbc.at[pl.ds(0, CH)], ocf.at[pl.ds(off, CH)])
            pltpu.sync_copy(sbw.at[pl.ds(0, CH)], owf.at[pl.ds(off, CH)])

        nch = nch + jnp.where(woff > 0, 1, 0)
        nch8 = ((nch + GRP2 - 1) // GRP2) * GRP2

        for i in range(CH // 16):
            sbr[pl.ds(i * 16, 16)] = dumr
            sbc[pl.ds(i * 16, 16)] = zi
            sbw[pl.ds(i * 16, 16)] = zf

        def pad_body(d, _, orf=orf, ocf=ocf, owf=owf, obase=obase, nch=nch):
            off = obase + (nch + d) * CH
            pltpu.sync_copy(sbr.at[pl.ds(0, CH)], orf.at[pl.ds(off, CH)])
            pltpu.sync_copy(sbc.at[pl.ds(0, CH)], ocf.at[pl.ds(off, CH)])
            pltpu.sync_copy(sbw.at[pl.ds(0, CH)], owf.at[pl.ds(off, CH)])
            return 0

        lax.fori_loop(0, nch8 - nch, pad_body, 0)

        cntb[...] = jnp.broadcast_to(nch8, (16,)).astype(jnp.int32)
        pltpu.sync_copy(cntb, cnts.at[j, wid, 0])


def _filter(r0, c0, w0, r1, c1, w1, tgt1d):
    SB = GRP * CH + CH + 96
    kfn = pl.kernel(
        _filter_body,
        out_type=[
            jax.ShapeDtypeStruct((NW * CAPE,), jnp.int32),
            jax.ShapeDtypeStruct((NW * CAPE,), jnp.int32),
            jax.ShapeDtypeStruct((NW * CAPE,), jnp.float32),
            jax.ShapeDtypeStruct((NW * CAPE,), jnp.int32),
            jax.ShapeDtypeStruct((NW * CAPE,), jnp.int32),
            jax.ShapeDtypeStruct((NW * CAPE,), jnp.float32),
            jax.ShapeDtypeStruct((NET, NW, 1, 16), jnp.int32),
        ],
        mesh=_mesh(),
        compiler_params=pltpu.CompilerParams(needs_layout_passes=False),
        scratch_types=[
            pltpu.VMEM((NPAD,), jnp.float32),
            pltpu.VMEM((NTPAD,), jnp.int32),
            pltpu.VMEM((GRP, CH), jnp.int32),
            pltpu.VMEM((GRP, CH), jnp.int32),
            pltpu.VMEM((GRP, CH), jnp.float32),
            pltpu.VMEM((SB,), jnp.int32),
            pltpu.VMEM((SB,), jnp.int32),
            pltpu.VMEM((SB,), jnp.float32),
            pltpu.VMEM((16,), jnp.int32),
        ],
    )
    return kfn(r0, c0, w0, r1, c1, w1, tgt1d)


def _layer2_body(hsrc, fr0, fc0, fw0, fr1, fc1, fw1, cnts, scl, z2d, p_out,
                 idf, icf, wff, idxr2, wsf, rows, svec, cntb, acc, gsem, ssem):
    c = lax.axis_index("c")
    s = lax.axis_index("s")
    wid = s * NC + c

    def g_desc(ch, k, slot):
        return pltpu.make_async_copy(
            hsrc.at[ch].at[icf.at[pl.ds(k * CH, CH)]], rows.at[slot],
            gsem.at[slot])

    def s_desc(k, slot):
        return pltpu.make_async_copy(
            rows.at[slot], acc.at[idxr2.at[k]], ssem.at[slot])

    for ch in range(NCH):
        pltpu.sync_copy(z2d.at[pl.ds(s * NSLICE, NSLICE)],
                        acc.at[pl.ds(s * NSLICE, NSLICE)])
        plsc.subcore_barrier()

        for j, (rref, cref, wref) in enumerate(
                ((fr0, fc0, fw0), (fr1, fc1, fw1))):
            pltpu.sync_copy(scl.at[ch, j], svec)
            pltpu.sync_copy(cnts.at[j, wid, 0], cntb)
            ng = cntb[pl.ds(0, 16)][0] // GRP2

            def grp_body(g, _, rref=rref, cref=cref, wref=wref, ch=ch):
                base = wid * CAPE + g * (GRP2 * CH)
                pltpu.sync_copy(rref.at[pl.ds(base, GRP2 * CH)], idf)
                pltpu.sync_copy(cref.at[pl.ds(base, GRP2 * CH)], icf)
                pltpu.sync_copy(wref.at[pl.ds(base, GRP2 * CH)], wff)

                def mv_body(v, _):
                    sl = pl.ds(v * 16, 16)
                    idxr2[v // (CH // 16), pl.ds((v % (CH // 16)) * 16, 16)] = (
                        idf[sl])
                    wsf[sl] = wff[sl] * svec[...]
                    return 0

                lax.fori_loop(0, GRP2 * CH // 16, mv_body, 0)

                g_desc(ch, 0, 0).start()

                def pair_body(kk, _, ch=ch):
                    for i in range(NBUF):
                        k = NBUF * kk + i
                        oi = 1 - i

                        @pl.when(k + 1 < GRP2)
                        def _(k=k, oi=oi):
                            @pl.when(k >= 1)
                            def _():
                                s_desc(k, oi).wait()
                            g_desc(ch, k + 1, oi).start()

                        g_desc(ch, k, i).wait()

                        def edge_body(e, _, k=k, i=i):
                            widx = jnp.full((16,), k * CH + e, jnp.int32)
                            w16 = plsc.load_gather(wsf, [widx])
                            for sub in range(D // 16):
                                rows[i, e, pl.ds(sub * 16, 16)] = (
                                    rows[i, e, pl.ds(sub * 16, 16)] * w16
                                )
                            return 0

                        lax.fori_loop(0, CH, edge_body, 0)
                        pltpu.async_copy(rows.at[i], acc.at[idxr2.at[k]],
                                         ssem.at[i], add=True)
                    return 0

                lax.fori_loop(0, GRP2 // NBUF, pair_body, 0)
                s_desc(GRP2 - 2, 0).wait()
                s_desc(GRP2 - 1, 1).wait()
                return 0

            lax.fori_loop(0, ng, grp_body, 0)

        plsc.subcore_barrier()
        pltpu.sync_copy(acc.at[pl.ds(s * NSLICE, NSLICE)],
                        p_out.at[c, ch, pl.ds(s * NSLICE, NSLICE)])
        plsc.subcore_barrier()


def _layer2(hsrc, fr0, fc0, fw0, fr1, fc1, fw1, cnts, scl, z2d):
    kfn = pl.kernel(
        _layer2_body,
        out_type=jax.ShapeDtypeStruct((NC, NCH, NPAD, D), jnp.float32),
        mesh=_mesh(),
        compiler_params=pltpu.CompilerParams(needs_layout_passes=False),
        scratch_types=[
            pltpu.VMEM((GRP2 * CH,), jnp.int32),
            pltpu.VMEM((GRP2 * CH,), jnp.int32),
            pltpu.VMEM((GRP2 * CH,), jnp.float32),
            pltpu.VMEM((GRP2, CH), jnp.int32),
            pltpu.VMEM((GRP2 * CH,), jnp.float32),
            pltpu.VMEM((NBUF, CH, D), jnp.float32),
            pltpu.VMEM((16,), jnp.float32),
            pltpu.VMEM((16,), jnp.int32),
            pltpu.VMEM_SHARED((NPAD, D), jnp.float32),
            pltpu.SemaphoreType.DMA((NBUF,)),
            pltpu.SemaphoreType.DMA((NBUF,)),
        ],
    )
    return kfn(hsrc, fr0, fc0, fw0, fr1, fc1, fw1, cnts, scl, z2d)


def _combine_body(a_ref, b_ref, o_ref):
    o_ref[0] = a_ref[0, 0] + b_ref[0, 0]


def _combine(p):
    BN = 1280
    grid = (NCH, NPAD // BN)
    return pl.pallas_call(
        _combine_body,
        grid=grid,
        in_specs=[
            pl.BlockSpec((1, 1, BN, D), lambda c, i: (0, c, i, 0)),
            pl.BlockSpec((1, 1, BN, D), lambda c, i: (1, c, i, 0)),
        ],
        out_specs=pl.BlockSpec((1, BN, D), lambda c, i: (c, i, 0)),
        out_shape=jax.ShapeDtypeStruct((NCH, NPAD, D), jnp.float32),
    )(p, p)


def _tgt_body(p2, hx, tgt2d, gout, tbuf, bufa, bufb, bufx):
    c = lax.axis_index("c")
    s = lax.axis_index("s")
    wid = s * NC + c
    pltpu.sync_copy(tgt2d.at[wid, 0], tbuf)
    for ch in range(NCH):
        pltpu.sync_copy(p2.at[0, ch].at[tbuf], bufa)
        pltpu.sync_copy(p2.at[1, ch].at[tbuf], bufb)
        pltpu.sync_copy(hx.at[ch].at[tbuf], bufx)

        def row_body(r, _):
            for sub in range(D // 16):
                sl = pl.ds(sub * 16, 16)
                v = BETA * bufx[r, sl] + (1.0 - BETA) * (bufa[r, sl] + bufb[r, sl])
                bufa[r, sl] = jnp.maximum(v, 0.0)
            return 0

        lax.fori_loop(0, NTPAD // NW, row_body, 0)
        pltpu.sync_copy(
            bufa, gout.at[pl.ds(wid * (NTPAD // NW), NTPAD // NW),
                          pl.ds(ch * D, D)])


def _tgt(p2, hx, tgt2d):
    TW = NTPAD // NW
    kfn = pl.kernel(
        _tgt_body,
        out_type=jax.ShapeDtypeStruct((NTPAD, NCH * D), jnp.float32),
        mesh=_mesh(),
        compiler_params=pltpu.CompilerParams(needs_layout_passes=False),
        scratch_types=[
            pltpu.VMEM((TW,), jnp.int32),
            pltpu.VMEM((TW, D), jnp.float32),
            pltpu.VMEM((TW, D), jnp.float32),
            pltpu.VMEM((TW, D), jnp.float32),
        ],
    )
    return kfn(p2, hx, tgt2d)


def _head_body(g_ref, w1_ref, b1_ref, w2_ref, b2_ref, y_ref):
    h = jnp.dot(g_ref[...], w1_ref[...], preferred_element_type=jnp.float32)
    h = jnp.maximum(h + b1_ref[...], 0.0)
    y = jnp.dot(h, w2_ref[...], preferred_element_type=jnp.float32)
    y_ref[...] = y + b2_ref[...]


def _head(g, W1, b1, W2, b2):
    BN = 256
    grid = (NTPAD // BN,)
    return pl.pallas_call(
        _head_body,
        grid=grid,
        in_specs=[
            pl.BlockSpec((BN, NCH * D), lambda i: (i, 0)),
            pl.BlockSpec((NCH * D, D), lambda i: (0, 0)),
            pl.BlockSpec((1, D), lambda i: (0, 0)),
            pl.BlockSpec((D, NCLASS), lambda i: (0, 0)),
            pl.BlockSpec((1, NCLASS), lambda i: (0, 0)),
        ],
        out_specs=pl.BlockSpec((BN, NCLASS), lambda i: (i, 0)),
        out_shape=jax.ShapeDtypeStruct((NTPAD, NCLASS), jnp.float32),
    )(g, W1, b1, W2, b2)


def kernel(x, edge_index_0, edge_value_0, edge_index_1, edge_value_1,
           target_x, Ws, filt0, filt1, W1, b1, W2, b2):
    pe = EPAD - E
    r0 = jnp.pad(edge_index_0[0], (0, pe), constant_values=N).reshape(EROWS, CH)
    c0 = jnp.pad(edge_index_0[1], (0, pe)).reshape(EROWS, CH)
    r1 = jnp.pad(edge_index_1[0], (0, pe), constant_values=N).reshape(EROWS, CH)
    c1 = jnp.pad(edge_index_1[1], (0, pe)).reshape(EROWS, CH)
    v0 = jnp.pad(edge_value_0, (0, pe)).reshape(EROWS, CH)
    v1 = jnp.pad(edge_value_1, (0, pe)).reshape(EROWS, CH)

    xpad = jnp.pad(x, ((0, NPAD - N), (0, 0)))
    zflat = jnp.zeros((NPAD * 2,), jnp.float32)
    z2d = jnp.zeros((NPAD, D), jnp.float32)
    tgt1d = jnp.pad(target_x, (0, NTPAD - NT), constant_values=N)
    tgt2d = tgt1d.reshape(NW, 1, NTPAD // NW)

    hx, scl = _proj(xpad, Ws, filt0, filt1)
    evn0, evn1 = _norm(r0, v0, r1, v1, zflat)
    p1 = _layer(hx, r0, c0, evn0, r1, c1, evn1, scl[0], z2d)
    h1 = _combine(p1)
    fr0, fc0, fw0, fr1, fc1, fw1, cnts = _filter(
        r0, c0, evn0, r1, c1, evn1, tgt1d)
    p2 = _layer2(h1, fr0, fc0, fw0, fr1, fc1, fw1, cnts, scl[1], z2d)
    g = _tgt(p2, hx, tgt2d)
    y = _head(g, W1, b1.reshape(1, D), W2, b2.reshape(1, NCLASS))
    return y[:NT]


# speedup vs baseline: 1.2941x; 1.2941x over previous
import jax
import jax.numpy as jnp
from jax import lax
from jax.experimental import pallas as pl
from jax.experimental.pallas import tpu as pltpu
from jax.experimental.pallas import tpu_sc as plsc

N = 10000
NPAD = 10240
E = 320000
EPAD = 327680
D = 128
NCH = 2
NET = 2
NCLASS = 16
NT = 2000
NTPAD = 2048
BETA = 0.5

NC = 2
NS = 16
NW = NC * NS
CH = 64
EROWS = EPAD // CH
ROWS_PER_W = EROWS // NW
GRP = 32
NGRP = ROWS_PER_W // GRP
ROWS_PER_T = EROWS // NS
NGRP_DEG = ROWS_PER_T // GRP
NSLICE = NPAD // NS


def _mesh():
    return plsc.VectorSubcoreMesh(core_axis_name="c", subcore_axis_name="s")


def _proj_body(x_ref, ws_ref, f0_ref, f1_ref, h_ref, scl_ref):
    ci = pl.program_id(0)
    bi = pl.program_id(1)
    h_ref[0] = jnp.dot(x_ref[...], ws_ref[0], preferred_element_type=jnp.float32)

    @pl.when(jnp.logical_and(ci == 0, bi == 0))
    def _():
        for l, fr in enumerate((f0_ref, f1_ref)):
            fw = jax.nn.softmax(fr[...], axis=1)
            scl_ref[l] = jnp.broadcast_to(fw[:, :, None], (NCH, NET, 16))


def _proj(xpad, Ws, filt0, filt1):
    BN = 1280
    grid = (NCH, NPAD // BN)
    return pl.pallas_call(
        _proj_body,
        grid=grid,
        in_specs=[
            pl.BlockSpec((BN, D), lambda c, i: (i, 0)),
            pl.BlockSpec((1, D, D), lambda c, i: (c, 0, 0)),
            pl.BlockSpec((NCH, NET), lambda c, i: (0, 0)),
            pl.BlockSpec((NCH, NET), lambda c, i: (0, 0)),
        ],
        out_specs=[
            pl.BlockSpec((1, BN, D), lambda c, i: (c, i, 0)),
            pl.BlockSpec((2, NCH, NET, 16), lambda c, i: (0, 0, 0, 0)),
        ],
        out_shape=[
            jax.ShapeDtypeStruct((NCH, NPAD, D), jnp.float32),
            jax.ShapeDtypeStruct((2, NCH, NET, 16), jnp.float32),
        ],
    )(xpad, Ws, filt0, filt1)


def _norm_body(r0, v0, r1, v1, zflat, evn0, evn1,
               rowb, evb, deg0, deg1, sumb, invb, inv0, inv1, evnb,
               spdeg, spinv):
    c = lax.axis_index("c")
    s = lax.axis_index("s")
    wid = s * NC + c

    for j, (rref, vref, dref) in enumerate(((r0, v0, deg0), (r1, v1, deg1))):
        pltpu.sync_copy(zflat.at[pl.ds(0, NPAD)], dref)

        def grp_body(g, _, rref=rref, vref=vref, dref=dref):
            base = s * ROWS_PER_T + g * GRP
            pltpu.sync_copy(rref.at[pl.ds(base, GRP)], rowb)
            pltpu.sync_copy(vref.at[pl.ds(base, GRP)], evb)

            def row_body(r, _):
                for k in range(CH // 16):
                    rv = rowb[r, pl.ds(k * 16, 16)]
                    ev = evb[r, pl.ds(k * 16, 16)]
                    plsc.addupdate_scatter(dref, [rv], ev)
                return 0

            lax.fori_loop(0, GRP, row_body, 0)
            return 0

        lax.fori_loop(0, NGRP_DEG, grp_body, 0)
        pltpu.sync_copy(dref, spdeg.at[j, s, 0])

    plsc.subcore_barrier()

    for j in range(NET):
        pltpu.sync_copy(spdeg.at[j, :, 0, pl.ds(s * NSLICE, NSLICE)], sumb)

        def inv_body(g, _):
            acc = sumb[0, pl.ds(g * 16, 16)]
            for r in range(1, NS):
                acc = acc + sumb[r, pl.ds(g * 16, 16)]
            inv = jnp.where(acc > 0.0, 1.0 / acc, 0.0)
            invb[pl.ds(g * 16, 16)] = inv
            return 0

        lax.fori_loop(0, NSLICE // 16, inv_body, 0)
        pltpu.sync_copy(invb, spinv.at[j, pl.ds(s * NSLICE, NSLICE)])

    plsc.subcore_barrier()
    pltpu.sync_copy(spinv.at[0], inv0)
    pltpu.sync_copy(spinv.at[1], inv1)

    for rref, vref, iref, oref in ((r0, v0, inv0, evn0), (r1, v1, inv1, evn1)):
        def grp2_body(g, _, rref=rref, vref=vref, iref=iref, oref=oref):
            base = wid * ROWS_PER_W + g * GRP
            pltpu.sync_copy(rref.at[pl.ds(base, GRP)], rowb)
            pltpu.sync_copy(vref.at[pl.ds(base, GRP)], evb)

            def row_body(r, _):
                for k in range(CH // 16):
                    rv = rowb[r, pl.ds(k * 16, 16)]
                    ev = evb[r, pl.ds(k * 16, 16)]
                    iv = plsc.load_gather(iref, [rv])
                    evnb[r, pl.ds(k * 16, 16)] = ev * iv
                return 0

            lax.fori_loop(0, GRP, row_body, 0)
            pltpu.sync_copy(evnb, oref.at[pl.ds(base, GRP)])
            return 0

        lax.fori_loop(0, NGRP, grp2_body, 0)


def _norm(r0, v0, r1, v1, zflat):
    kfn = pl.kernel(
        _norm_body,
        out_type=[
            jax.ShapeDtypeStruct((EROWS, CH), jnp.float32),
            jax.ShapeDtypeStruct((EROWS, CH), jnp.float32),
        ],
        mesh=_mesh(),
        compiler_params=pltpu.CompilerParams(needs_layout_passes=False),
        scratch_types=[
            pltpu.VMEM((GRP, CH), jnp.int32),
            pltpu.VMEM((GRP, CH), jnp.float32),
            pltpu.VMEM((NPAD,), jnp.float32),
            pltpu.VMEM((NPAD,), jnp.float32),
            pltpu.VMEM((NS, NSLICE), jnp.float32),
            pltpu.VMEM((NSLICE,), jnp.float32),
            pltpu.VMEM((NPAD,), jnp.float32),
            pltpu.VMEM((NPAD,), jnp.float32),
            pltpu.VMEM((GRP, CH), jnp.float32),
            pltpu.VMEM_SHARED((NET, NS, 1, NPAD), jnp.float32),
            pltpu.VMEM_SHARED((NET, NPAD), jnp.float32),
        ],
    )
    return kfn(r0, v0, r1, v1, zflat)


NBUF = 4


def _layer_body(hsrc, r0, c0, w0, r1, c1, w1, scl, z2d, p_out,
                idxr, idxc, wb, wsf, rows, svec, acc, gsem, ssem):
    c = lax.axis_index("c")
    s = lax.axis_index("s")
    wid = s * NC + c

    def g_desc(ch, k, slot):
        return pltpu.make_async_copy(
            hsrc.at[ch].at[idxc.at[k]], rows.at[slot], gsem.at[slot])

    def s_desc(k, slot):
        return pltpu.make_async_copy(
            rows.at[slot], acc.at[idxr.at[k]], ssem.at[slot])

    for ch in range(NCH):
        pltpu.sync_copy(z2d.at[pl.ds(s * NSLICE, NSLICE)],
                        acc.at[pl.ds(s * NSLICE, NSLICE)])
        plsc.subcore_barrier()

        for j, (rref, cref, wref) in enumerate(((r0, c0, w0), (r1, c1, w1))):
            pltpu.sync_copy(scl.at[ch, j], svec)

            def grp_body(g, _, rref=rref, cref=cref, wref=wref, ch=ch):
                base = wid * ROWS_PER_W + g * GRP
                pltpu.sync_copy(rref.at[pl.ds(base, GRP)], idxr)
                pltpu.sync_copy(cref.at[pl.ds(base, GRP)], idxc)
                pltpu.sync_copy(wref.at[pl.ds(base, GRP)], wb)

                def swr_body(r, _):
                    for q in range(CH // 16):
                        wsf[pl.ds(r * CH + q * 16, 16)] = (
                            wb[r, pl.ds(q * 16, 16)] * svec[...]
                        )
                    return 0

                lax.fori_loop(0, GRP, swr_body, 0)

                g_desc(ch, 0, 0).start()
                g_desc(ch, 1, 1).start()

                def ring_body(kk, _, ch=ch):
                    for i in range(NBUF):
                        k = NBUF * kk + i
                        j2 = (i + 2) % NBUF

                        @pl.when(k + 2 < GRP)
                        def _(k=k, j2=j2):
                            g_desc(ch, k + 2, j2).start()

                        g_desc(ch, k, i).wait()

                        def eg_body(g16, _, k=k, i=i):
                            for e in range(16):
                                widx = jnp.full(
                                    (16,), k * CH + g16 * 16 + e, jnp.int32)
                                w16 = plsc.load_gather(wsf, [widx])
                                r = g16 * 16 + e
                                for sub in range(D // 16):
                                    rows[i, r, pl.ds(sub * 16, 16)] = (
                                        rows[i, r, pl.ds(sub * 16, 16)] * w16
                                    )
                            return 0

                        lax.fori_loop(0, CH // 16, eg_body, 0)

                        @pl.when(k >= 1)
                        def _(k=k, i=i):
                            s_desc(k - 1, (i - 1) % NBUF).wait()
                        pltpu.async_copy(rows.at[i], acc.at[idxr.at[k]],
                                         ssem.at[i], add=True)
                    return 0

                lax.fori_loop(0, GRP // NBUF, ring_body, 0)
                s_desc(GRP - 1, (GRP - 1) % NBUF).wait()
                return 0

            lax.fori_loop(0, NGRP, grp_body, 0)

        plsc.subcore_barrier()
        pltpu.sync_copy(acc.at[pl.ds(s * NSLICE, NSLICE)],
                        p_out.at[c, ch, pl.ds(s * NSLICE, NSLICE)])
        plsc.subcore_barrier()


def _layer(hsrc, r0, c0, w0, r1, c1, w1, scl, z2d):
    kfn = pl.kernel(
        _layer_body,
        out_type=jax.ShapeDtypeStruct((NC, NCH, NPAD, D), jnp.float32),
        mesh=_mesh(),
        compiler_params=pltpu.CompilerParams(needs_layout_passes=False),
        scratch_types=[
            pltpu.VMEM((GRP, CH), jnp.int32),
            pltpu.VMEM((GRP, CH), jnp.int32),
            pltpu.VMEM((GRP, CH), jnp.float32),
            pltpu.VMEM((GRP * CH,), jnp.float32),
            pltpu.VMEM((NBUF, CH, D), jnp.float32),
            pltpu.VMEM((16,), jnp.float32),
            pltpu.VMEM_SHARED((NPAD, D), jnp.float32),
            pltpu.SemaphoreType.DMA((NBUF,)),
            pltpu.SemaphoreType.DMA((NBUF,)),
        ],
    )
    return kfn(hsrc, r0, c0, w0, r1, c1, w1, scl, z2d)


CAPC = ROWS_PER_W
CAPE = CAPC * CH
GRP2 = 8


def _filter_body(r0, c0, w0, r1, c1, w1, tgt1d,
                 fr0, fc0, fw0, fr1, fc1, fw1, cnts,
                 flag, tball, rowb, colb, wvb, sbr, sbc, sbw, cntb):
    c = lax.axis_index("c")
    s = lax.axis_index("s")
    wid = s * NC + c
    ones = jnp.ones((16,), jnp.float32)

    def z_body(i, _):
        flag[pl.ds(i * 16, 16)] = jnp.zeros((16,), jnp.float32)
        return 0

    lax.fori_loop(0, NPAD // 16, z_body, 0)
    pltpu.sync_copy(tgt1d, tball)

    def t_body(i, _):
        tv = tball[pl.ds(i * 16, 16)]
        plsc.store_scatter(flag, [tv], ones)
        return 0

    lax.fori_loop(0, NTPAD // 16, t_body, 0)

    dumr = jnp.full((16,), N, jnp.int32)
    zi = jnp.zeros((16,), jnp.int32)
    zf = jnp.zeros((16,), jnp.float32)

    for j, (rref, cref, wref, orf, ocf, owf) in enumerate((
            (r0, c0, w0, fr0, fc0, fw0), (r1, c1, w1, fr1, fc1, fw1))):
        obase = wid * CAPE

        def grp_body(g, carry, rref=rref, cref=cref, wref=wref,
                     orf=orf, ocf=ocf, owf=owf, obase=obase):
            woff, nch = carry
            base = wid * ROWS_PER_W + g * GRP
            pltpu.sync_copy(rref.at[pl.ds(base, GRP)], rowb)
            pltpu.sync_copy(cref.at[pl.ds(base, GRP)], colb)
            pltpu.sync_copy(wref.at[pl.ds(base, GRP)], wvb)

            def row_body(r, woff):
                for k in range(CH // 16):
                    rv = rowb[r, pl.ds(k * 16, 16)]
                    cv = colb[r, pl.ds(k * 16, 16)]
                    wv = wvb[r, pl.ds(k * 16, 16)]
                    fv = plsc.load_gather(flag, [rv])
                    m = fv > 0.5
                    plsc.store_compressed(sbr.at[pl.ds(woff, 16)], rv, mask=m)
                    plsc.store_compressed(sbc.at[pl.ds(woff, 16)], cv, mask=m)
                    plsc.store_compressed(sbw.at[pl.ds(woff, 16)], wv, mask=m)
                    pc = plsc.all_reduce_population_count(m)
                    woff = woff + jnp.max(pc, axis=0)
                return woff

            woff = lax.fori_loop(0, GRP, row_body, woff)

            ndump = woff // CH

            def dump_body(d, _):
                off = obase + (nch + d) * CH
                pltpu.sync_copy(sbr.at[pl.ds(d * CH, CH)], orf.at[pl.ds(off, CH)])
                pltpu.sync_copy(sbc.at[pl.ds(d * CH, CH)], ocf.at[pl.ds(off, CH)])
                pltpu.sync_copy(sbw.at[pl.ds(d * CH, CH)], owf.at[pl.ds(off, CH)])
                return 0

            lax.fori_loop(0, ndump, dump_body, 0)
            rem = woff - ndump * CH

            for i in range(CH // 16 + 1):
                sr = sbr[pl.ds(ndump * CH + i * 16, 16)]
                sc_ = sbc[pl.ds(ndump * CH + i * 16, 16)]
                sw = sbw[pl.ds(ndump * CH + i * 16, 16)]
                sbr[pl.ds(i * 16, 16)] = sr
                sbc[pl.ds(i * 16, 16)] = sc_
                sbw[pl.ds(i * 16, 16)] = sw
            return (rem, nch + ndump)

        woff, nch = lax.fori_loop(0, NGRP, grp_body, (0, 0))

        for i in range(CH // 16 + 1):
            sbr[pl.ds(woff + i * 16, 16)] = dumr
            sbc[pl.ds(woff + i * 16, 16)] = zi
            sbw[pl.ds(woff + i * 16, 16)] = zf

        @pl.when(woff > 0)
        def _(nch=nch, obase=obase, orf=orf, ocf=ocf, owf=owf):
            off = obase + nch * CH
            pltpu.sync_copy(sbr.at[pl.ds(0, CH)], orf.at[pl.ds(off, CH)])
            pltpu.sync_copy(sbc.at[pl.ds(0, CH)], ocf.at[pl.ds(off, CH)])
            pltpu.sync_copy(sbw.at[pl.ds(0, CH)], owf.at[pl.ds(off, CH)])

        nch = nch + jnp.where(woff > 0, 1, 0)
        nch8 = ((nch + GRP2 - 1) // GRP2) * GRP2

        for i in range(CH // 16):
            sbr[pl.ds(i * 16, 16)] = dumr
            sbc[pl.ds(i * 16, 16)] = zi
            sbw[pl.ds(i * 16, 16)] = zf

        def pad_body(d, _, orf=orf, ocf=ocf, owf=owf, obase=obase, nch=nch):
            off = obase + (nch + d) * CH
            pltpu.sync_copy(sbr.at[pl.ds(0, CH)], orf.at[pl.ds(off, CH)])
            pltpu.sync_copy(sbc.at[pl.ds(0, CH)], ocf.at[pl.ds(off, CH)])
            pltpu.sync_copy(sbw.at[pl.ds(0, CH)], owf.at[pl.ds(off, CH)])
            return 0

        lax.fori_loop(0, nch8 - nch, pad_body, 0)

        cntb[...] = jnp.broadcast_to(nch8, (16,)).astype(jnp.int32)
        pltpu.sync_copy(cntb, cnts.at[j, wid, 0])


def _filter(r0, c0, w0, r1, c1, w1, tgt1d):
    SB = GRP * CH + CH + 96
    kfn = pl.kernel(
        _filter_body,
        out_type=[
            jax.ShapeDtypeStruct((NW * CAPE,), jnp.int32),
            jax.ShapeDtypeStruct((NW * CAPE,), jnp.int32),
            jax.ShapeDtypeStruct((NW * CAPE,), jnp.float32),
            jax.ShapeDtypeStruct((NW * CAPE,), jnp.int32),
            jax.ShapeDtypeStruct((NW * CAPE,), jnp.int32),
            jax.ShapeDtypeStruct((NW * CAPE,), jnp.float32),
            jax.ShapeDtypeStruct((NET, NW, 1, 16), jnp.int32),
        ],
        mesh=_mesh(),
        compiler_params=pltpu.CompilerParams(needs_layout_passes=False),
        scratch_types=[
            pltpu.VMEM((NPAD,), jnp.float32),
            pltpu.VMEM((NTPAD,), jnp.int32),
            pltpu.VMEM((GRP, CH), jnp.int32),
            pltpu.VMEM((GRP, CH), jnp.int32),
            pltpu.VMEM((GRP, CH), jnp.float32),
            pltpu.VMEM((SB,), jnp.int32),
            pltpu.VMEM((SB,), jnp.int32),
            pltpu.VMEM((SB,), jnp.float32),
            pltpu.VMEM((16,), jnp.int32),
        ],
    )
    return kfn(r0, c0, w0, r1, c1, w1, tgt1d)


def _layer2_body(hsrc, fr0, fc0, fw0, fr1, fc1, fw1, cnts, scl, z2d, p_out,
                 idf, icf, wff, idxr2, wsf, rows, svec, cntb, acc, gsem, ssem):
    c = lax.axis_index("c")
    s = lax.axis_index("s")
    wid = s * NC + c

    def g_desc(ch, k, slot):
        return pltpu.make_async_copy(
            hsrc.at[ch].at[icf.at[pl.ds(k * CH, CH)]], rows.at[slot],
            gsem.at[slot])

    def s_desc(k, slot):
        return pltpu.make_async_copy(
            rows.at[slot], acc.at[idxr2.at[k]], ssem.at[slot])

    for ch in range(NCH):
        pltpu.sync_copy(z2d.at[pl.ds(s * NSLICE, NSLICE)],
                        acc.at[pl.ds(s * NSLICE, NSLICE)])
        plsc.subcore_barrier()

        for j, (rref, cref, wref) in enumerate(
                ((fr0, fc0, fw0), (fr1, fc1, fw1))):
            pltpu.sync_copy(scl.at[ch, j], svec)
            pltpu.sync_copy(cnts.at[j, wid, 0], cntb)
            ng = cntb[pl.ds(0, 16)][0] // GRP2

            def grp_body(g, _, rref=rref, cref=cref, wref=wref, ch=ch):
                base = wid * CAPE + g * (GRP2 * CH)
                pltpu.sync_copy(rref.at[pl.ds(base, GRP2 * CH)], idf)
                pltpu.sync_copy(cref.at[pl.ds(base, GRP2 * CH)], icf)
                pltpu.sync_copy(wref.at[pl.ds(base, GRP2 * CH)], wff)

                def swr2_body(v, _):
                    sl = pl.ds(v * 16, 16)
                    wsf[sl] = wff[sl] * svec[...]
                    return 0

                lax.fori_loop(0, GRP2 * CH // 16, swr2_body, 0)

                def mv_body(v, _):
                    sl = pl.ds(v * 16, 16)
                    idxr2[v // (CH // 16), pl.ds((v % (CH // 16)) * 16, 16)] = (
                        idf[sl])
                    return 0

                lax.fori_loop(0, GRP2 * CH // 16, mv_body, 0)

                g_desc(ch, 0, 0).start()
                g_desc(ch, 1, 1).start()

                def ring_body(kk, _, ch=ch):
                    for i in range(NBUF):
                        k = NBUF * kk + i
                        j2 = (i + 2) % NBUF

                        @pl.when(k + 2 < GRP2)
                        def _(k=k, j2=j2):
                            g_desc(ch, k + 2, j2).start()

                        g_desc(ch, k, i).wait()

                        def eg_body(g16, _, k=k, i=i):
                            for e in range(16):
                                widx = jnp.full(
                                    (16,), k * CH + g16 * 16 + e, jnp.int32)
                                w16 = plsc.load_gather(wsf, [widx])
                                r = g16 * 16 + e
                                for sub in range(D // 16):
                                    rows[i, r, pl.ds(sub * 16, 16)] = (
                                        rows[i, r, pl.ds(sub * 16, 16)] * w16
                                    )
                            return 0

                        lax.fori_loop(0, CH // 16, eg_body, 0)

                        @pl.when(k >= 1)
                        def _(k=k, i=i):
                            s_desc(k - 1, (i - 1) % NBUF).wait()
                        pltpu.async_copy(rows.at[i], acc.at[idxr2.at[k]],
                                         ssem.at[i], add=True)
                    return 0

                lax.fori_loop(0, GRP2 // NBUF, ring_body, 0)
                s_desc(GRP2 - 1, (GRP2 - 1) % NBUF).wait()
                return 0

            lax.fori_loop(0, ng, grp_body, 0)

        plsc.subcore_barrier()
        pltpu.sync_copy(acc.at[pl.ds(s * NSLICE, NSLICE)],
                        p_out.at[c, ch, pl.ds(s * NSLICE, NSLICE)])
        plsc.subcore_barrier()


def _layer2(hsrc, fr0, fc0, fw0, fr1, fc1, fw1, cnts, scl, z2d):
    kfn = pl.kernel(
        _layer2_body,
        out_type=jax.ShapeDtypeStruct((NC, NCH, NPAD, D), jnp.float32),
        mesh=_mesh(),
        compiler_params=pltpu.CompilerParams(needs_layout_passes=False),
        scratch_types=[
            pltpu.VMEM((GRP2 * CH,), jnp.int32),
            pltpu.VMEM((GRP2 * CH,), jnp.int32),
            pltpu.VMEM((GRP2 * CH,), jnp.float32),
            pltpu.VMEM((GRP2, CH), jnp.int32),
            pltpu.VMEM((GRP2 * CH,), jnp.float32),
            pltpu.VMEM((NBUF, CH, D), jnp.float32),
            pltpu.VMEM((16,), jnp.float32),
            pltpu.VMEM((16,), jnp.int32),
            pltpu.VMEM_SHARED((NPAD, D), jnp.float32),
            pltpu.SemaphoreType.DMA((NBUF,)),
            pltpu.SemaphoreType.DMA((NBUF,)),
        ],
    )
    return kfn(hsrc, fr0, fc0, fw0, fr1, fc1, fw1, cnts, scl, z2d)


def _combine_body(a_ref, b_ref, o_ref):
    o_ref[0] = a_ref[0, 0] + b_ref[0, 0]


def _combine(p):
    BN = 1280
    grid = (NCH, NPAD // BN)
    return pl.pallas_call(
        _combine_body,
        grid=grid,
        in_specs=[
            pl.BlockSpec((1, 1, BN, D), lambda c, i: (0, c, i, 0)),
            pl.BlockSpec((1, 1, BN, D), lambda c, i: (1, c, i, 0)),
        ],
        out_specs=pl.BlockSpec((1, BN, D), lambda c, i: (c, i, 0)),
        out_shape=jax.ShapeDtypeStruct((NCH, NPAD, D), jnp.float32),
    )(p, p)


def _tgt_body(p2, hx, tgt2d, gout, tbuf, bufa, bufb, bufx):
    c = lax.axis_index("c")
    s = lax.axis_index("s")
    wid = s * NC + c
    pltpu.sync_copy(tgt2d.at[wid, 0], tbuf)
    for ch in range(NCH):
        pltpu.sync_copy(p2.at[0, ch].at[tbuf], bufa)
        pltpu.sync_copy(p2.at[1, ch].at[tbuf], bufb)
        pltpu.sync_copy(hx.at[ch].at[tbuf], bufx)

        def row_body(r, _):
            for sub in range(D // 16):
                sl = pl.ds(sub * 16, 16)
                v = BETA * bufx[r, sl] + (1.0 - BETA) * (bufa[r, sl] + bufb[r, sl])
                bufa[r, sl] = jnp.maximum(v, 0.0)
            return 0

        lax.fori_loop(0, NTPAD // NW, row_body, 0)
        pltpu.sync_copy(
            bufa, gout.at[pl.ds(wid * (NTPAD // NW), NTPAD // NW),
                          pl.ds(ch * D, D)])


def _tgt(p2, hx, tgt2d):
    TW = NTPAD // NW
    kfn = pl.kernel(
        _tgt_body,
        out_type=jax.ShapeDtypeStruct((NTPAD, NCH * D), jnp.float32),
        mesh=_mesh(),
        compiler_params=pltpu.CompilerParams(needs_layout_passes=False),
        scratch_types=[
            pltpu.VMEM((TW,), jnp.int32),
            pltpu.VMEM((TW, D), jnp.float32),
            pltpu.VMEM((TW, D), jnp.float32),
            pltpu.VMEM((TW, D), jnp.float32),
        ],
    )
    return kfn(p2, hx, tgt2d)


def _head_body(g_ref, w1_ref, b1_ref, w2_ref, b2_ref, y_ref):
    h = jnp.dot(g_ref[...], w1_ref[...], preferred_element_type=jnp.float32)
    h = jnp.maximum(h + b1_ref[...], 0.0)
    y = jnp.dot(h, w2_ref[...], preferred_element_type=jnp.float32)
    y_ref[...] = y + b2_ref[...]


def _head(g, W1, b1, W2, b2):
    BN = 256
    grid = (NTPAD // BN,)
    return pl.pallas_call(
        _head_body,
        grid=grid,
        in_specs=[
            pl.BlockSpec((BN, NCH * D), lambda i: (i, 0)),
            pl.BlockSpec((NCH * D, D), lambda i: (0, 0)),
            pl.BlockSpec((1, D), lambda i: (0, 0)),
            pl.BlockSpec((D, NCLASS), lambda i: (0, 0)),
            pl.BlockSpec((1, NCLASS), lambda i: (0, 0)),
        ],
        out_specs=pl.BlockSpec((BN, NCLASS), lambda i: (i, 0)),
        out_shape=jax.ShapeDtypeStruct((NTPAD, NCLASS), jnp.float32),
    )(g, W1, b1, W2, b2)


def kernel(x, edge_index_0, edge_value_0, edge_index_1, edge_value_1,
           target_x, Ws, filt0, filt1, W1, b1, W2, b2):
    pe = EPAD - E
    r0 = jnp.pad(edge_index_0[0], (0, pe), constant_values=N + 8).reshape(EROWS, CH)
    c0 = jnp.pad(edge_index_0[1], (0, pe)).reshape(EROWS, CH)
    r1 = jnp.pad(edge_index_1[0], (0, pe), constant_values=N + 8).reshape(EROWS, CH)
    c1 = jnp.pad(edge_index_1[1], (0, pe)).reshape(EROWS, CH)
    v0 = jnp.pad(edge_value_0, (0, pe)).reshape(EROWS, CH)
    v1 = jnp.pad(edge_value_1, (0, pe)).reshape(EROWS, CH)

    xpad = jnp.pad(x, ((0, NPAD - N), (0, 0)))
    zflat = jnp.zeros((NPAD * 2,), jnp.float32)
    z2d = jnp.zeros((NPAD, D), jnp.float32)
    tgt1d = jnp.pad(target_x, (0, NTPAD - NT), constant_values=N)
    tgt2d = tgt1d.reshape(NW, 1, NTPAD // NW)

    hx, scl = _proj(xpad, Ws, filt0, filt1)
    evn0, evn1 = _norm(r0, v0, r1, v1, zflat)
    p1 = _layer(hx, r0, c0, evn0, r1, c1, evn1, scl[0], z2d)
    h1 = _combine(p1)
    fr0, fc0, fw0, fr1, fc1, fw1, cnts = _filter(
        r0, c0, evn0, r1, c1, evn1, tgt1d)
    p2 = _layer2(h1, fr0, fc0, fw0, fr1, fc1, fw1, cnts, scl[1], z2d)
    g = _tgt(p2, hx, tgt2d)
    y = _head(g, W1, b1.reshape(1, D), W2, b2.reshape(1, NCLASS))
    return y[:NT]
